# baseline (speedup 1.0000x reference)
import functools

import jax
import jax.numpy as jnp
from jax import lax
from jax.experimental import pallas as pl
from jax.experimental.pallas import tpu as pltpu
from jax.experimental.pallas import tpu_sc as plsc


def _sc_gather(input_ids, W):
    L = input_ids.shape[0]
    D = W.shape[1]
    info = plsc.get_sparse_core_info()
    rows_per_w = L // info.num_subcores
    mesh = plsc.VectorSubcoreMesh(
        core_axis_name="c", subcore_axis_name="s", num_cores=1)

    @functools.partial(
        pl.kernel,
        mesh=mesh,
        out_type=jax.ShapeDtypeStruct((L, D), jnp.float32),
        scratch_types=[
            pltpu.VMEM((rows_per_w,), jnp.int32),
            pltpu.VMEM((rows_per_w, D), jnp.float32),
            pltpu.SemaphoreType.DMA,
        ],
    )
    def gather_kernel(ids_hbm, table_hbm, out_hbm, idx_v, rows_v, sem):
        base = lax.axis_index("s") * rows_per_w
        pltpu.sync_copy(ids_hbm.at[pl.ds(base, rows_per_w)], idx_v)
        pltpu.async_copy(table_hbm.at[idx_v], rows_v, sem).wait()
        pltpu.sync_copy(rows_v, out_hbm.at[pl.ds(base, rows_per_w)])

    return gather_kernel(input_ids, W)


def kernel(x, input_ids, W):
    B = x.shape[0]
    L = input_ids.shape[0]
    D = W.shape[1]
    BB = 8

    emb = _sc_gather(input_ids, W)

    def body(emb_ref, out_ref):
        out_ref[...] = jnp.broadcast_to(emb_ref[...][None], (BB, L, D))

    bcast = pl.pallas_call(
        body,
        grid=(B // BB,),
        in_specs=[pl.BlockSpec((L, D), lambda b: (0, 0))],
        out_specs=pl.BlockSpec((BB, L, D), lambda b: (b, 0, 0)),
        out_shape=jax.ShapeDtypeStruct((B, L, D), jnp.float32),
    )(W[:L])
    return bcast.at[0, 0, 0].add(emb[0, 0] * 0)


# speedup vs baseline: 1.0654x; 1.0654x over previous
import functools

import jax
import jax.numpy as jnp
from jax import lax
from jax.experimental import pallas as pl
from jax.experimental.pallas import tpu as pltpu
from jax.experimental.pallas import tpu_sc as plsc


def _sc_gather(input_ids, W):
    L = input_ids.shape[0]
    D = W.shape[1]
    info = plsc.get_sparse_core_info()
    rows_per_w = L // info.num_subcores
    mesh = plsc.VectorSubcoreMesh(
        core_axis_name="c", subcore_axis_name="s", num_cores=1)

    @functools.partial(
        pl.kernel,
        mesh=mesh,
        out_type=jax.ShapeDtypeStruct((L, D), jnp.float32),
        scratch_types=[
            pltpu.VMEM((rows_per_w,), jnp.int32),
            pltpu.VMEM((rows_per_w, D), jnp.float32),
            pltpu.SemaphoreType.DMA,
        ],
    )
    def gather_kernel(ids_hbm, table_hbm, out_hbm, idx_v, rows_v, sem):
        base = lax.axis_index("s") * rows_per_w
        pltpu.sync_copy(ids_hbm.at[pl.ds(base, rows_per_w)], idx_v)
        pltpu.async_copy(table_hbm.at[idx_v], rows_v, sem).wait()
        pltpu.sync_copy(rows_v, out_hbm.at[pl.ds(base, rows_per_w)])

    return gather_kernel(input_ids, W)


def _tc_broadcast(emb, B):
    L, D = emb.shape
    BB = 8

    def body(emb_ref, out_ref):
        out_ref[...] = jnp.broadcast_to(emb_ref[...][None], (BB, L, D))

    return pl.pallas_call(
        body,
        grid=(B // BB,),
        in_specs=[pl.BlockSpec((L, D), lambda b: (0, 0))],
        out_specs=pl.BlockSpec((BB, L, D), lambda b: (b, 0, 0)),
        out_shape=jax.ShapeDtypeStruct((B, L, D), jnp.float32),
    )(emb)


def kernel(x, input_ids, W):
    B = x.shape[0]
    emb = _sc_gather(input_ids, W)
    return _tc_broadcast(emb, B)
